# baseline (speedup 1.0000x reference)
import jax
import jax.numpy as jnp
from jax import lax
from jax.experimental import pallas as pl

_K = 64
_ROWS_PER_BLOCK = 8
_INT_MIN32 = -2147483648


def _bitval(b):
    return jnp.int32(_INT_MIN32) if b == 31 else jnp.int32(1 << b)


def _topk_mask_body(x_ref, o_ref):
    x = x_ref[...]
    r, n = x.shape

    raw = lax.bitcast_convert_type(x, jnp.int32)
    ikey = jnp.where(raw >= 0, raw, raw ^ jnp.int32(0x7FFFFFFF))

    def _lane_sum(y):
        part = jnp.sum(y.reshape(r, y.shape[1] // 128, 128), axis=1)
        return jnp.sum(part, axis=1, keepdims=True)

    def count_ge(t):
        return _lane_sum((ikey >= t).astype(jnp.int32))

    m = jnp.max(ikey.reshape(r, n // 64, 64), axis=1)
    u = jnp.max(m, axis=1, keepdims=True)

    lo0 = jnp.min(m, axis=1, keepdims=True)
    cl0 = count_ge(lo0)
    hi0 = u + 1
    ch0 = jnp.zeros((r, 1), jnp.int32)

    def active(lo, hi, cl):
        return (cl > _K) & (hi - 1 > lo)

    def probe(state, bisect):
        lo, hi, cl, ch = state
        act = active(lo, hi, cl)
        wf = lo.astype(jnp.float32) * (-1.0) + hi.astype(jnp.float32)
        frac = (cl - _K).astype(jnp.float32) / jnp.maximum(
            (cl - ch).astype(jnp.float32), 1.0)
        stepf = jnp.clip(wf * frac, 1.0, jnp.maximum(wf - 1.0, 1.0))
        mid_i = lo + stepf.astype(jnp.int32)
        mid_b = (lo & hi) + ((lo ^ hi) >> 1)
        mid = jnp.where(bisect, mid_b, mid_i)
        mid = jnp.maximum(lo + 1, jnp.minimum(mid, hi - 1))
        c = count_ge(mid)
        up = c >= _K
        lo = jnp.where(act & up, mid, lo)
        cl = jnp.where(act & up, c, cl)
        hi = jnp.where(act & ~up, mid, hi)
        ch = jnp.where(act & ~up, c, ch)
        return lo, hi, cl, ch

    state = (lo0, hi0, cl0, ch0)
    for _ in range(6):
        state = probe(state, bisect=False)

    def loop_cond(s):
        it, lo, hi, cl, ch = s
        return jnp.any(active(lo, hi, cl))

    def loop_body(s):
        it, lo, hi, cl, ch = s
        lo, hi, cl, ch = probe((lo, hi, cl, ch), bisect=(it % 2 == 1))
        return it + 1, lo, hi, cl, ch

    _, lo, hi, cl, ch = lax.while_loop(
        loop_cond, loop_body, (jnp.int32(0),) + state)

    t = lo
    ties_any = jnp.any(cl > _K)

    @pl.when(jnp.logical_not(ties_any))
    def _no_ties():
        o_ref[...] = jnp.where(ikey >= t, jnp.maximum(x, 0.0), 0.0)

    @pl.when(ties_any)
    def _with_ties():
        gt = ikey > t
        eq = ikey == t
        cnt_gt = _lane_sum(gt.astype(jnp.int32))
        need = _K - cnt_gt
        idx = lax.broadcasted_iota(jnp.int32, x.shape, 1)
        ipref = jnp.zeros((r, 1), dtype=jnp.int32)
        for b in range(14, -1, -1):
            tr = ipref + jnp.int32(1 << b)
            c = _lane_sum((eq & (idx < tr)).astype(jnp.int32))
            ipref = jnp.where(c < need, tr, ipref)
        mask = gt | (eq & (idx <= ipref))
        o_ref[...] = jnp.where(mask, jnp.maximum(x, 0.0), 0.0)


def kernel(x):
    rows, n = x.shape
    rb = _ROWS_PER_BLOCK
    return pl.pallas_call(
        _topk_mask_body,
        grid=(rows // rb,),
        in_specs=[pl.BlockSpec((rb, n), lambda i: (i, 0))],
        out_specs=pl.BlockSpec((rb, n), lambda i: (i, 0)),
        out_shape=jax.ShapeDtypeStruct(x.shape, x.dtype),
    )(x)


# speedup vs baseline: 1.5430x; 1.5430x over previous
import jax
import jax.numpy as jnp
from jax import lax
from jax.experimental import pallas as pl

_K = 64
_ROWS_PER_BLOCK = 16
_INT_MIN32 = -2147483648


def _bitval(b):
    return jnp.int32(_INT_MIN32) if b == 31 else jnp.int32(1 << b)


def _topk_mask_body(x_ref, o_ref):
    x = x_ref[...]
    r, n = x.shape

    raw = lax.bitcast_convert_type(x, jnp.int32)
    ikey = jnp.where(raw >= 0, raw, raw ^ jnp.int32(0x7FFFFFFF))

    def _lane_sum(y):
        part = jnp.sum(y.reshape(r, y.shape[1] // 128, 128), axis=1)
        return jnp.sum(part, axis=1, keepdims=True)

    def count_ge(t):
        return _lane_sum((ikey >= t).astype(jnp.int32))

    m = jnp.max(ikey.reshape(r, n // 512, 512), axis=1)
    u = jnp.max(m, axis=1, keepdims=True)

    lp = jnp.full((r, 1), _INT_MIN32, jnp.int32)
    for b in range(31, 15, -1):
        tr = lp ^ _bitval(b)
        c = _lane_sum((m >= tr).astype(jnp.int32))
        lp = jnp.where(c >= _K, tr, lp)

    lo0 = lp
    cl0 = count_ge(lo0)
    hi0 = u + 1
    ch0 = jnp.zeros((r, 1), jnp.int32)

    def active(lo, hi, cl):
        return (cl > _K) & (hi - 1 > lo)

    def loop_cond(state):
        it, lo, hi, cl, ch = state
        return jnp.any(active(lo, hi, cl))

    def loop_body(state):
        it, lo, hi, cl, ch = state
        act = active(lo, hi, cl)
        wf = lo.astype(jnp.float32) * (-1.0) + hi.astype(jnp.float32)
        frac = (cl - _K).astype(jnp.float32) / jnp.maximum(
            (cl - ch).astype(jnp.float32), 1.0)
        stepf = jnp.clip(wf * frac, 1.0, jnp.maximum(wf - 1.0, 1.0))
        mid_i = lo + stepf.astype(jnp.int32)
        mid_b = (lo & hi) + ((lo ^ hi) >> 1)
        mid = jnp.where(it % 2 == 0, mid_i, mid_b)
        mid = jnp.maximum(lo + 1, jnp.minimum(mid, hi - 1))
        c = count_ge(mid)
        up = c >= _K
        lo = jnp.where(act & up, mid, lo)
        cl = jnp.where(act & up, c, cl)
        hi = jnp.where(act & ~up, mid, hi)
        ch = jnp.where(act & ~up, c, ch)
        return it + 1, lo, hi, cl, ch

    _, lo, hi, cl, ch = lax.while_loop(
        loop_cond, loop_body,
        (jnp.int32(0), lo0, hi0, cl0, ch0))

    t = lo
    ties_any = jnp.any(cl > _K)

    @pl.when(jnp.logical_not(ties_any))
    def _no_ties():
        o_ref[...] = jnp.where(ikey >= t, jnp.maximum(x, 0.0), 0.0)

    @pl.when(ties_any)
    def _with_ties():
        gt = ikey > t
        eq = ikey == t
        cnt_gt = _lane_sum(gt.astype(jnp.int32))
        need = _K - cnt_gt
        idx = lax.broadcasted_iota(jnp.int32, x.shape, 1)
        ipref = jnp.zeros((r, 1), dtype=jnp.int32)
        for b in range(14, -1, -1):
            tr = ipref + jnp.int32(1 << b)
            c = _lane_sum((eq & (idx < tr)).astype(jnp.int32))
            ipref = jnp.where(c < need, tr, ipref)
        mask = gt | (eq & (idx <= ipref))
        o_ref[...] = jnp.where(mask, jnp.maximum(x, 0.0), 0.0)


def kernel(x):
    rows, n = x.shape
    rb = _ROWS_PER_BLOCK
    return pl.pallas_call(
        _topk_mask_body,
        grid=(rows // rb,),
        in_specs=[pl.BlockSpec((rb, n), lambda i: (i, 0))],
        out_specs=pl.BlockSpec((rb, n), lambda i: (i, 0)),
        out_shape=jax.ShapeDtypeStruct(x.shape, x.dtype),
    )(x)


# speedup vs baseline: 1.5753x; 1.0210x over previous
import jax
import jax.numpy as jnp
from jax import lax
from jax.experimental import pallas as pl

_K = 64
_ROWS_PER_BLOCK = 32
_INT_MIN32 = -2147483648


def _bitval(b):
    return jnp.int32(_INT_MIN32) if b == 31 else jnp.int32(1 << b)


def _topk_mask_body(x_ref, o_ref):
    x = x_ref[...]
    r, n = x.shape

    raw = lax.bitcast_convert_type(x, jnp.int32)
    ikey = jnp.where(raw >= 0, raw, raw ^ jnp.int32(0x7FFFFFFF))

    def _lane_sum(y):
        part = jnp.sum(y.reshape(r, y.shape[1] // 128, 128), axis=1)
        return jnp.sum(part, axis=1, keepdims=True)

    def count_ge(t):
        return _lane_sum((ikey >= t).astype(jnp.int32))

    m = jnp.max(ikey.reshape(r, n // 512, 512), axis=1)
    u = jnp.max(m, axis=1, keepdims=True)

    lp = jnp.full((r, 1), _INT_MIN32, jnp.int32)
    for b in range(31, 15, -1):
        tr = lp ^ _bitval(b)
        c = _lane_sum((m >= tr).astype(jnp.int32))
        lp = jnp.where(c >= _K, tr, lp)

    lo0 = lp
    cl0 = count_ge(lo0)
    hi0 = u + 1
    ch0 = jnp.zeros((r, 1), jnp.int32)

    def active(lo, hi, cl):
        return (cl > _K) & (hi - 1 > lo)

    def loop_cond(state):
        it, lo, hi, cl, ch = state
        return jnp.any(active(lo, hi, cl))

    def loop_body(state):
        it, lo, hi, cl, ch = state
        act = active(lo, hi, cl)
        wf = lo.astype(jnp.float32) * (-1.0) + hi.astype(jnp.float32)
        frac = (cl - _K).astype(jnp.float32) / jnp.maximum(
            (cl - ch).astype(jnp.float32), 1.0)
        stepf = jnp.clip(wf * frac, 1.0, jnp.maximum(wf - 1.0, 1.0))
        mid_i = lo + stepf.astype(jnp.int32)
        mid_b = (lo & hi) + ((lo ^ hi) >> 1)
        mid = jnp.where(it % 2 == 0, mid_i, mid_b)
        mid = jnp.maximum(lo + 1, jnp.minimum(mid, hi - 1))
        c = count_ge(mid)
        up = c >= _K
        lo = jnp.where(act & up, mid, lo)
        cl = jnp.where(act & up, c, cl)
        hi = jnp.where(act & ~up, mid, hi)
        ch = jnp.where(act & ~up, c, ch)
        return it + 1, lo, hi, cl, ch

    _, lo, hi, cl, ch = lax.while_loop(
        loop_cond, loop_body,
        (jnp.int32(0), lo0, hi0, cl0, ch0))

    t = lo
    ties_any = jnp.any(cl > _K)

    @pl.when(jnp.logical_not(ties_any))
    def _no_ties():
        o_ref[...] = jnp.where(ikey >= t, jnp.maximum(x, 0.0), 0.0)

    @pl.when(ties_any)
    def _with_ties():
        gt = ikey > t
        eq = ikey == t
        cnt_gt = _lane_sum(gt.astype(jnp.int32))
        need = _K - cnt_gt
        idx = lax.broadcasted_iota(jnp.int32, x.shape, 1)
        ipref = jnp.zeros((r, 1), dtype=jnp.int32)
        for b in range(14, -1, -1):
            tr = ipref + jnp.int32(1 << b)
            c = _lane_sum((eq & (idx < tr)).astype(jnp.int32))
            ipref = jnp.where(c < need, tr, ipref)
        mask = gt | (eq & (idx <= ipref))
        o_ref[...] = jnp.where(mask, jnp.maximum(x, 0.0), 0.0)


def kernel(x):
    rows, n = x.shape
    rb = _ROWS_PER_BLOCK
    return pl.pallas_call(
        _topk_mask_body,
        grid=(rows // rb,),
        in_specs=[pl.BlockSpec((rb, n), lambda i: (i, 0))],
        out_specs=pl.BlockSpec((rb, n), lambda i: (i, 0)),
        out_shape=jax.ShapeDtypeStruct(x.shape, x.dtype),
    )(x)


# speedup vs baseline: 1.5915x; 1.0103x over previous
import jax
import jax.numpy as jnp
from jax import lax
from jax.experimental import pallas as pl

_K = 64
_ROWS_PER_BLOCK = 32
_INT_MIN32 = -2147483648


def _bitval(b):
    return jnp.int32(_INT_MIN32) if b == 31 else jnp.int32(1 << b)


def _topk_mask_body(x_ref, o_ref):
    x = x_ref[...]
    r, n = x.shape

    raw = lax.bitcast_convert_type(x, jnp.int32)
    ikey = jnp.where(raw >= 0, raw, raw ^ jnp.int32(0x7FFFFFFF))

    def _lane_sum(y):
        part = jnp.sum(y.reshape(r, y.shape[1] // 128, 128), axis=1)
        return jnp.sum(part, axis=1, keepdims=True)

    def count_ge(t):
        return _lane_sum((ikey >= t).astype(jnp.int32))

    m = jnp.max(ikey.reshape(r, n // 512, 512), axis=1)
    u = jnp.max(m, axis=1, keepdims=True)

    lp = jnp.full((r, 1), _INT_MIN32, jnp.int32)
    for b in range(31, 11, -1):
        tr = lp ^ _bitval(b)
        c = _lane_sum((m >= tr).astype(jnp.int32))
        lp = jnp.where(c >= _K, tr, lp)

    lo0 = lp
    cl0 = count_ge(lo0)
    hi0 = u + 1
    ch0 = jnp.zeros((r, 1), jnp.int32)

    def active(lo, hi, cl):
        return (cl > _K) & (hi - 1 > lo)

    def loop_cond(state):
        it, lo, hi, cl, ch = state
        return jnp.any(active(lo, hi, cl))

    def loop_body(state):
        it, lo, hi, cl, ch = state
        act = active(lo, hi, cl)
        wf = lo.astype(jnp.float32) * (-1.0) + hi.astype(jnp.float32)
        frac = (cl - _K).astype(jnp.float32) / jnp.maximum(
            (cl - ch).astype(jnp.float32), 1.0)
        stepf = jnp.clip(wf * frac, 1.0, jnp.maximum(wf - 1.0, 1.0))
        mid_i = lo + stepf.astype(jnp.int32)
        mid_b = (lo & hi) + ((lo ^ hi) >> 1)
        mid = jnp.where(it % 2 == 0, mid_i, mid_b)
        mid = jnp.maximum(lo + 1, jnp.minimum(mid, hi - 1))
        c = count_ge(mid)
        up = c >= _K
        lo = jnp.where(act & up, mid, lo)
        cl = jnp.where(act & up, c, cl)
        hi = jnp.where(act & ~up, mid, hi)
        ch = jnp.where(act & ~up, c, ch)
        return it + 1, lo, hi, cl, ch

    _, lo, hi, cl, ch = lax.while_loop(
        loop_cond, loop_body,
        (jnp.int32(0), lo0, hi0, cl0, ch0))

    t = lo
    ties_any = jnp.any(cl > _K)

    @pl.when(jnp.logical_not(ties_any))
    def _no_ties():
        o_ref[...] = jnp.where(ikey >= t, jnp.maximum(x, 0.0), 0.0)

    @pl.when(ties_any)
    def _with_ties():
        gt = ikey > t
        eq = ikey == t
        cnt_gt = _lane_sum(gt.astype(jnp.int32))
        need = _K - cnt_gt
        idx = lax.broadcasted_iota(jnp.int32, x.shape, 1)
        ipref = jnp.zeros((r, 1), dtype=jnp.int32)
        for b in range(14, -1, -1):
            tr = ipref + jnp.int32(1 << b)
            c = _lane_sum((eq & (idx < tr)).astype(jnp.int32))
            ipref = jnp.where(c < need, tr, ipref)
        mask = gt | (eq & (idx <= ipref))
        o_ref[...] = jnp.where(mask, jnp.maximum(x, 0.0), 0.0)


def kernel(x):
    rows, n = x.shape
    rb = _ROWS_PER_BLOCK
    return pl.pallas_call(
        _topk_mask_body,
        grid=(rows // rb,),
        in_specs=[pl.BlockSpec((rb, n), lambda i: (i, 0))],
        out_specs=pl.BlockSpec((rb, n), lambda i: (i, 0)),
        out_shape=jax.ShapeDtypeStruct(x.shape, x.dtype),
    )(x)
